# baseline (speedup 1.0000x reference)
import jax
import jax.numpy as jnp
from jax.experimental import pallas as pl
from jax.experimental.pallas import tpu as pltpu

D_MODEL = 768
NUM_EXPERTS = 8
ROUTED_DFF = 2304
SHARED_DFF = 768
SEQ = 2048

TILE = 128
NT = 40
GROWS = NT * TILE
GBLK = 512
NGB = GROWS // GBLK
CH = 768
NCH = ROUTED_DFF // CH
CTILE = 256


def _fiota(shape, dim):
    return jax.lax.broadcasted_iota(jnp.int32, shape, dim).astype(jnp.float32)


def _cumsum_rows(oh, tri):
    nblk = SEQ // GBLK
    outs = []
    carry = jnp.zeros((1, NUM_EXPERTS), jnp.float32)
    for b in range(nblk):
        blk = oh[b * GBLK:(b + 1) * GBLK, :]
        ex = jax.lax.dot_general(tri, blk.astype(jnp.bfloat16),
                                 (((1,), (0,)), ((), ())),
                                 preferred_element_type=jnp.float32)
        outs.append(ex + carry)
        carry = carry + jnp.sum(blk, axis=0, keepdims=True)
    return jnp.concatenate(outs, axis=0), carry


def _dispatch_kernel(x_ref, ln_scale_ref, ln_bias_ref, router_W_ref,
                     xg_ref, xb_out_ref, rt_ref, plan_ref,
                     xb_ref, posT_ref):
    p = pl.program_id(0)

    @pl.when(p == 0)
    def _route():
        x = x_ref[...]
        mu = jnp.mean(x, axis=1, keepdims=True)
        xc = x - mu
        var = jnp.mean(xc * xc, axis=1, keepdims=True)
        xn = xc * jax.lax.rsqrt(var + 1e-5)
        xn = xn * ln_scale_ref[...] + ln_bias_ref[...]
        xb = xn.astype(jnp.bfloat16)
        xb_ref[...] = xb
        xb_out_ref[...] = xb
        logits = jax.lax.dot_general(
            xb, router_W_ref[...].astype(jnp.bfloat16),
            (((1,), (1,)), ((), ())),
            preferred_element_type=jnp.float32)
        m = jnp.max(logits, axis=1, keepdims=True)
        ex = jnp.exp(logits - m)
        probs = ex / jnp.sum(ex, axis=1, keepdims=True)
        iota = _fiota(probs.shape, 1)
        p1 = jnp.max(probs, axis=1, keepdims=True)
        i1 = jnp.min(jnp.where(probs == p1, iota, NUM_EXPERTS), axis=1,
                     keepdims=True)
        masked = jnp.where(iota == i1, -1.0, probs)
        p2 = jnp.max(masked, axis=1, keepdims=True)
        i2 = jnp.min(jnp.where(masked == p2, iota, NUM_EXPERTS), axis=1,
                     keepdims=True)
        b = jnp.exp(p2 - p1)
        w1 = 1.0 / (1.0 + b)
        w2 = b / (1.0 + b)
        oh1 = (iota == i1).astype(jnp.float32)
        oh2 = (iota == i2).astype(jnp.float32)
        tri = (_fiota((GBLK, GBLK), 0) > _fiota((GBLK, GBLK), 1)
               ).astype(jnp.bfloat16)
        ex1, tot1 = _cumsum_rows(oh1, tri)
        ex2, tot2 = _cumsum_rows(oh2, tri)
        cnt = tot1 + tot2
        rank1 = jnp.sum(ex1 * oh1, axis=1, keepdims=True)
        rank2 = (jnp.sum(tot1 * oh2, axis=1, keepdims=True)
                 + jnp.sum(ex2 * oh2, axis=1, keepdims=True))
        ntiles = jnp.floor((cnt + (TILE - 1)) * (1.0 / TILE))
        sizes = ntiles * TILE
        tri8 = (_fiota((8, 8), 0) < _fiota((8, 8), 1)).astype(jnp.float32)
        offs = jax.lax.dot_general(sizes, tri8, (((1,), (0,)), ((), ())),
                                   precision=jax.lax.Precision.HIGHEST,
                                   preferred_element_type=jnp.float32)
        bounds = jax.lax.dot_general(
            ntiles,
            (_fiota((8, 8), 0) <= _fiota((8, 8), 1)).astype(jnp.float32),
            (((1,), (0,)), ((), ())),
            precision=jax.lax.Precision.HIGHEST,
            preferred_element_type=jnp.float32)
        pos1 = jnp.sum(offs * oh1, axis=1, keepdims=True) + rank1
        pos2 = jnp.sum(offs * oh2, axis=1, keepdims=True) + rank2
        rt = jnp.concatenate([pos1, pos2, w1, w2, i1, i2, jnp.zeros_like(p1),
                              jnp.zeros_like(p1)], axis=1)
        rt_ref[...] = rt
        P = jnp.concatenate(
            [pos1, pos2] + [jnp.zeros_like(pos1)] * 6, axis=1)
        posT_ref[...] = jnp.transpose(P, (1, 0))
        tstart = bounds - ntiles
        plan_ref[...] = jnp.concatenate(
            [tstart, bounds, jnp.zeros((6, 8), jnp.float32)], axis=0)

    rows = jnp.float32(GBLK) * p + _fiota((GBLK, 1), 0)
    pt = posT_ref[...]
    g1 = (pt[0:1, :] == rows).astype(jnp.bfloat16)
    g2 = (pt[1:2, :] == rows).astype(jnp.bfloat16)
    xg_ref[...] = jax.lax.dot_general(
        g1 + g2, xb_ref[...], (((1,), (0,)), ((), ())),
        preferred_element_type=jnp.float32).astype(jnp.bfloat16)


def _expert_kernel(ts_ref, tn_ref, xg_ref, gate_ref, up_ref, down_ref,
                   yg_ref):
    e = pl.program_id(0)
    c = pl.program_id(1)

    @pl.when(jnp.logical_and(e == 0, c == 0))
    def _init():
        yg_ref[...] = jnp.zeros_like(yg_ref)

    gw = gate_ref[0, 0].astype(jnp.bfloat16)
    uw = up_ref[0, 0].astype(jnp.bfloat16)
    dw = down_ref[0].astype(jnp.bfloat16)

    def body(jt, _):
        r0 = jt * TILE
        xt = xg_ref[pl.ds(r0, TILE), :]
        g = jax.lax.dot_general(xt, gw, (((1,), (1,)), ((), ())),
                                preferred_element_type=jnp.float32)
        u = jax.lax.dot_general(xt, uw, (((1,), (1,)), ((), ())),
                                preferred_element_type=jnp.float32)
        h = (g * jax.nn.sigmoid(g)) * u
        y = jax.lax.dot_general(h.astype(jnp.bfloat16), dw,
                                (((1,), (1,)), ((), ())),
                                preferred_element_type=jnp.float32)
        yg_ref[pl.ds(r0, TILE), :] += y
        return 0

    jax.lax.fori_loop(ts_ref[e], tn_ref[e], body, 0)


def _combine_kernel(rt_ref, yg_ref, xb_ref, sh_gate_ref, sh_up_ref,
                    sh_down_ref, out_ref):
    pos1 = rt_ref[:, 0:1]
    pos2 = rt_ref[:, 1:2]
    w1 = rt_ref[:, 2:3]
    w2 = rt_ref[:, 3:4]
    lanes = _fiota((CTILE, GROWS), 1)
    comb = (jnp.where(lanes == pos1, w1, 0.0)
            + jnp.where(lanes == pos2, w2, 0.0)).astype(jnp.bfloat16)
    routed = jax.lax.dot_general(comb, yg_ref[...].astype(jnp.bfloat16),
                                 (((1,), (0,)), ((), ())),
                                 preferred_element_type=jnp.float32)
    xb = xb_ref[...]
    sg = jax.lax.dot_general(xb, sh_gate_ref[0].astype(jnp.bfloat16),
                             (((1,), (1,)), ((), ())),
                             preferred_element_type=jnp.float32)
    su = jax.lax.dot_general(xb, sh_up_ref[0].astype(jnp.bfloat16),
                             (((1,), (1,)), ((), ())),
                             preferred_element_type=jnp.float32)
    sh = (sg * jax.nn.sigmoid(sg)) * su
    ys = jax.lax.dot_general(sh.astype(jnp.bfloat16),
                             sh_down_ref[...].astype(jnp.bfloat16),
                             (((1,), (1,)), ((), ())),
                             preferred_element_type=jnp.float32)
    out_ref[...] = routed + ys


@jax.jit
def kernel(x, ln_scale, ln_bias, router_W, shared_gate_up_W, shared_down_W,
           expert_gate_up_W, expert_down_W):
    B, S, D = x.shape
    x2 = x.reshape(S, D)
    ln_scale2 = ln_scale.reshape(1, D)
    ln_bias2 = ln_bias.reshape(1, D)

    xg, xb, rt, plan = pl.pallas_call(
        _dispatch_kernel,
        grid=(NGB,),
        in_specs=[
            pl.BlockSpec((S, D), lambda p: (0, 0)),
            pl.BlockSpec((1, D), lambda p: (0, 0)),
            pl.BlockSpec((1, D), lambda p: (0, 0)),
            pl.BlockSpec((NUM_EXPERTS, D), lambda p: (0, 0)),
        ],
        out_specs=[
            pl.BlockSpec((GBLK, D), lambda p: (p, 0)),
            pl.BlockSpec((S, D), lambda p: (0, 0)),
            pl.BlockSpec((S, 8), lambda p: (0, 0)),
            pl.BlockSpec((8, 8), lambda p: (0, 0)),
        ],
        out_shape=[
            jax.ShapeDtypeStruct((GROWS, D), jnp.bfloat16),
            jax.ShapeDtypeStruct((S, D), jnp.bfloat16),
            jax.ShapeDtypeStruct((S, 8), jnp.float32),
            jax.ShapeDtypeStruct((8, 8), jnp.float32),
        ],
        scratch_shapes=[
            pltpu.VMEM((S, D), jnp.bfloat16),
            pltpu.VMEM((8, S), jnp.float32),
        ],
        compiler_params=pltpu.CompilerParams(
            dimension_semantics=("arbitrary",)),
    )(x2, ln_scale2, ln_bias2, router_W)

    tstart = plan[0].astype(jnp.int32)
    tend = plan[1].astype(jnp.int32)

    gu4 = expert_gate_up_W.reshape(NUM_EXPERTS, 2 * NCH, CH, D)
    grid_spec = pltpu.PrefetchScalarGridSpec(
        num_scalar_prefetch=2,
        grid=(NUM_EXPERTS, NCH),
        in_specs=[
            pl.BlockSpec((GROWS, D), lambda e, c, ts, tn: (0, 0)),
            pl.BlockSpec((1, 1, CH, D), lambda e, c, ts, tn: (e, c, 0, 0)),
            pl.BlockSpec((1, 1, CH, D),
                         lambda e, c, ts, tn: (e, NCH + c, 0, 0)),
            pl.BlockSpec((1, D, CH), lambda e, c, ts, tn: (e, 0, c)),
        ],
        out_specs=pl.BlockSpec((GROWS, D), lambda e, c, ts, tn: (0, 0)),
    )
    yg = pl.pallas_call(
        _expert_kernel,
        grid_spec=grid_spec,
        out_shape=jax.ShapeDtypeStruct((GROWS, D), jnp.float32),
        compiler_params=pltpu.CompilerParams(
            dimension_semantics=("arbitrary", "arbitrary")),
    )(tstart, tend, xg, gu4, gu4, expert_down_W)

    shW = shared_gate_up_W.reshape(2, SHARED_DFF, D)
    out = pl.pallas_call(
        _combine_kernel,
        grid=(S // CTILE,),
        in_specs=[
            pl.BlockSpec((CTILE, 8), lambda t: (t, 0)),
            pl.BlockSpec((GROWS, D), lambda t: (0, 0)),
            pl.BlockSpec((CTILE, D), lambda t: (t, 0)),
            pl.BlockSpec((1, SHARED_DFF, D), lambda t: (0, 0, 0)),
            pl.BlockSpec((1, SHARED_DFF, D), lambda t: (1, 0, 0)),
            pl.BlockSpec((D, SHARED_DFF), lambda t: (0, 0)),
        ],
        out_specs=pl.BlockSpec((CTILE, D), lambda t: (t, 0)),
        out_shape=jax.ShapeDtypeStruct((S, D), jnp.float32),
        compiler_params=pltpu.CompilerParams(
            dimension_semantics=("arbitrary",)),
    )(rt, yg, xb, shW, shW, shared_down_W)
    return out.reshape(B, S, D)


_ORIG = kernel


# speedup vs baseline: 1.2752x; 1.2752x over previous
import jax
import jax.numpy as jnp
from jax.experimental import pallas as pl
from jax.experimental.pallas import tpu as pltpu
from jax.experimental.pallas import tpu_sc as plsc

D_MODEL = 768
NUM_EXPERTS = 8
ROUTED_DFF = 2304
SHARED_DFF = 768
SEQ = 2048

TILE = 256
NT = 24
GROWS = NT * TILE
GBLK = 512
NGB = GROWS // GBLK
CH = 768
NCH = ROUTED_DFF // CH
CTILE = 256
APW = (2 * SEQ) // 32


def _fiota(shape, dim):
    return jax.lax.broadcasted_iota(jnp.int32, shape, dim).astype(jnp.float32)


def _cumsum_rows(oh, tri):
    nblk = SEQ // GBLK
    outs = []
    carry = jnp.zeros((1, NUM_EXPERTS), jnp.float32)
    for b in range(nblk):
        blk = oh[b * GBLK:(b + 1) * GBLK, :]
        ex = jax.lax.dot_general(tri, blk.astype(jnp.bfloat16),
                                 (((1,), (0,)), ((), ())),
                                 preferred_element_type=jnp.float32)
        outs.append(ex + carry)
        carry = carry + jnp.sum(blk, axis=0, keepdims=True)
    return jnp.concatenate(outs, axis=0), carry


def _dispatch_kernel(x_ref, ln_scale_ref, ln_bias_ref, router_W_ref,
                     xn_out_ref, xb_out_ref, rt_ref, plan_ref):
    if True:
        x = x_ref[...]
        mu = jnp.mean(x, axis=1, keepdims=True)
        xc = x - mu
        var = jnp.mean(xc * xc, axis=1, keepdims=True)
        xn = xc * jax.lax.rsqrt(var + 1e-5)
        xn = xn * ln_scale_ref[...] + ln_bias_ref[...]
        xn_out_ref[...] = xn
        xb = xn.astype(jnp.bfloat16)
        xb_out_ref[...] = xb
        logits = jax.lax.dot_general(
            xb, router_W_ref[...].astype(jnp.bfloat16),
            (((1,), (1,)), ((), ())),
            preferred_element_type=jnp.float32)
        m = jnp.max(logits, axis=1, keepdims=True)
        ex = jnp.exp(logits - m)
        probs = ex / jnp.sum(ex, axis=1, keepdims=True)
        iota = _fiota(probs.shape, 1)
        p1 = jnp.max(probs, axis=1, keepdims=True)
        i1 = jnp.min(jnp.where(probs == p1, iota, NUM_EXPERTS), axis=1,
                     keepdims=True)
        masked = jnp.where(iota == i1, -1.0, probs)
        p2 = jnp.max(masked, axis=1, keepdims=True)
        i2 = jnp.min(jnp.where(masked == p2, iota, NUM_EXPERTS), axis=1,
                     keepdims=True)
        b = jnp.exp(p2 - p1)
        w1 = 1.0 / (1.0 + b)
        w2 = b / (1.0 + b)
        oh1 = (iota == i1).astype(jnp.float32)
        oh2 = (iota == i2).astype(jnp.float32)
        tri = (_fiota((GBLK, GBLK), 0) > _fiota((GBLK, GBLK), 1)
               ).astype(jnp.bfloat16)
        ex1, tot1 = _cumsum_rows(oh1, tri)
        ex2, tot2 = _cumsum_rows(oh2, tri)
        cnt = tot1 + tot2
        rank1 = jnp.sum(ex1 * oh1, axis=1, keepdims=True)
        rank2 = (jnp.sum(tot1 * oh2, axis=1, keepdims=True)
                 + jnp.sum(ex2 * oh2, axis=1, keepdims=True))
        ntiles = jnp.floor((cnt + (TILE - 1)) * (1.0 / TILE))
        sizes = ntiles * TILE
        tri8 = (_fiota((8, 8), 0) < _fiota((8, 8), 1)).astype(jnp.float32)
        offs = jax.lax.dot_general(sizes, tri8, (((1,), (0,)), ((), ())),
                                   precision=jax.lax.Precision.HIGHEST,
                                   preferred_element_type=jnp.float32)
        bounds = jax.lax.dot_general(
            ntiles,
            (_fiota((8, 8), 0) <= _fiota((8, 8), 1)).astype(jnp.float32),
            (((1,), (0,)), ((), ())),
            precision=jax.lax.Precision.HIGHEST,
            preferred_element_type=jnp.float32)
        pos1 = jnp.sum(offs * oh1, axis=1, keepdims=True) + rank1
        pos2 = jnp.sum(offs * oh2, axis=1, keepdims=True) + rank2
        rt = jnp.concatenate([pos1, pos2, w1, w2, i1, i2, jnp.zeros_like(p1),
                              jnp.zeros_like(p1)], axis=1)
        rt_ref[...] = rt
        tstart = bounds - ntiles
        plan_ref[...] = jnp.concatenate(
            [tstart, bounds, jnp.zeros((6, 8), jnp.float32)], axis=0)


def _sc_scatter_kernel(xn_hbm, idx_hbm, xg_hbm, idx_v, rows_v, sem):
    wid = (jax.lax.axis_index("s") * plsc.get_sparse_core_info().num_cores
           + jax.lax.axis_index("c"))
    base = wid * APW
    tbase = jax.lax.rem(base, SEQ)
    pltpu.sync_copy(idx_hbm.at[pl.ds(base, APW)], idx_v)
    pltpu.sync_copy(xn_hbm.at[pl.ds(tbase, APW), :], rows_v)
    pltpu.async_copy(rows_v, xg_hbm.at[idx_v], sem).wait()


def _expert_kernel(ts_ref, tn_ref, xg_ref, gate_ref, up_ref, down_ref,
                   yg_ref):
    e = pl.program_id(0)
    c = pl.program_id(1)

    @pl.when(jnp.logical_and(e == 0, c == 0))
    def _init():
        yg_ref[...] = jnp.zeros_like(yg_ref)

    gw = gate_ref[0, 0].astype(jnp.bfloat16)
    uw = up_ref[0, 0].astype(jnp.bfloat16)
    dw = down_ref[0].astype(jnp.bfloat16)

    def body(jt, _):
        r0 = jt * TILE
        xt = xg_ref[pl.ds(r0, TILE), :].astype(jnp.bfloat16)
        g = jax.lax.dot_general(xt, gw, (((1,), (1,)), ((), ())),
                                preferred_element_type=jnp.float32)
        u = jax.lax.dot_general(xt, uw, (((1,), (1,)), ((), ())),
                                preferred_element_type=jnp.float32)
        h = (g * jax.nn.sigmoid(g)) * u
        y = jax.lax.dot_general(h.astype(jnp.bfloat16), dw,
                                (((1,), (1,)), ((), ())),
                                preferred_element_type=jnp.float32)
        yg_ref[pl.ds(r0, TILE), :] += y
        return 0

    jax.lax.fori_loop(ts_ref[e], tn_ref[e], body, 0)


def _combine_kernel(rt_ref, yg_ref, xb_ref, sh_gate_ref, sh_up_ref,
                    sh_down_ref, out_ref):
    pos1 = rt_ref[:, 0:1]
    pos2 = rt_ref[:, 1:2]
    w1 = rt_ref[:, 2:3]
    w2 = rt_ref[:, 3:4]
    lanes = _fiota((CTILE, GROWS), 1)
    comb = (jnp.where(lanes == pos1, w1, 0.0)
            + jnp.where(lanes == pos2, w2, 0.0)).astype(jnp.bfloat16)
    ygv = yg_ref[...]
    ygv = jnp.where(jnp.abs(ygv) <= 3.0e38, ygv, 0.0)
    routed = jax.lax.dot_general(comb, ygv.astype(jnp.bfloat16),
                                 (((1,), (0,)), ((), ())),
                                 preferred_element_type=jnp.float32)
    xb = xb_ref[...]
    sg = jax.lax.dot_general(xb, sh_gate_ref[0].astype(jnp.bfloat16),
                             (((1,), (1,)), ((), ())),
                             preferred_element_type=jnp.float32)
    su = jax.lax.dot_general(xb, sh_up_ref[0].astype(jnp.bfloat16),
                             (((1,), (1,)), ((), ())),
                             preferred_element_type=jnp.float32)
    sh = (sg * jax.nn.sigmoid(sg)) * su
    ys = jax.lax.dot_general(sh.astype(jnp.bfloat16),
                             sh_down_ref[...].astype(jnp.bfloat16),
                             (((1,), (1,)), ((), ())),
                             preferred_element_type=jnp.float32)
    out_ref[...] = routed + ys


@jax.jit
def kernel(x, ln_scale, ln_bias, router_W, shared_gate_up_W, shared_down_W,
           expert_gate_up_W, expert_down_W):
    B, S, D = x.shape
    x2 = x.reshape(S, D)
    ln_scale2 = ln_scale.reshape(1, D)
    ln_bias2 = ln_bias.reshape(1, D)

    xn, xb, rt, plan = pl.pallas_call(
        _dispatch_kernel,
        grid=(1,),
        in_specs=[
            pl.BlockSpec((S, D), lambda p: (0, 0)),
            pl.BlockSpec((1, D), lambda p: (0, 0)),
            pl.BlockSpec((1, D), lambda p: (0, 0)),
            pl.BlockSpec((NUM_EXPERTS, D), lambda p: (0, 0)),
        ],
        out_specs=[
            pl.BlockSpec((S, D), lambda p: (0, 0)),
            pl.BlockSpec((S, D), lambda p: (0, 0)),
            pl.BlockSpec((S, 8), lambda p: (0, 0)),
            pl.BlockSpec((8, 8), lambda p: (0, 0)),
        ],
        out_shape=[
            jax.ShapeDtypeStruct((S, D), jnp.float32),
            jax.ShapeDtypeStruct((S, D), jnp.bfloat16),
            jax.ShapeDtypeStruct((S, 8), jnp.float32),
            jax.ShapeDtypeStruct((8, 8), jnp.float32),
        ],
        compiler_params=pltpu.CompilerParams(
            dimension_semantics=("arbitrary",)),
    )(x2, ln_scale2, ln_bias2, router_W)

    idx = jnp.concatenate([rt[:, 0], rt[:, 1]], axis=0).astype(jnp.int32)
    mesh = plsc.VectorSubcoreMesh(core_axis_name="c", subcore_axis_name="s")
    xg = pl.kernel(
        _sc_scatter_kernel,
        mesh=mesh,
        out_type=jax.ShapeDtypeStruct((GROWS, D), jnp.float32),
        scratch_types=[
            pltpu.VMEM((APW,), jnp.int32),
            pltpu.VMEM((APW, D), jnp.float32),
            pltpu.SemaphoreType.DMA,
        ],
    )(xn, idx)

    tstart = plan[0].astype(jnp.int32)
    tend = plan[1].astype(jnp.int32)

    gu4 = expert_gate_up_W.reshape(NUM_EXPERTS, 2 * NCH, CH, D)
    grid_spec = pltpu.PrefetchScalarGridSpec(
        num_scalar_prefetch=2,
        grid=(NUM_EXPERTS, NCH),
        in_specs=[
            pl.BlockSpec((GROWS, D), lambda e, c, ts, tn: (0, 0)),
            pl.BlockSpec((1, 1, CH, D), lambda e, c, ts, tn: (e, c, 0, 0)),
            pl.BlockSpec((1, 1, CH, D),
                         lambda e, c, ts, tn: (e, NCH + c, 0, 0)),
            pl.BlockSpec((1, D, CH), lambda e, c, ts, tn: (e, 0, c)),
        ],
        out_specs=pl.BlockSpec((GROWS, D), lambda e, c, ts, tn: (0, 0)),
    )
    yg = pl.pallas_call(
        _expert_kernel,
        grid_spec=grid_spec,
        out_shape=jax.ShapeDtypeStruct((GROWS, D), jnp.float32),
        compiler_params=pltpu.CompilerParams(
            dimension_semantics=("arbitrary", "arbitrary")),
    )(tstart, tend, xg, gu4, gu4, expert_down_W)

    shW = shared_gate_up_W.reshape(2, SHARED_DFF, D)
    out = pl.pallas_call(
        _combine_kernel,
        grid=(S // CTILE,),
        in_specs=[
            pl.BlockSpec((CTILE, 8), lambda t: (t, 0)),
            pl.BlockSpec((GROWS, D), lambda t: (0, 0)),
            pl.BlockSpec((CTILE, D), lambda t: (t, 0)),
            pl.BlockSpec((1, SHARED_DFF, D), lambda t: (0, 0, 0)),
            pl.BlockSpec((1, SHARED_DFF, D), lambda t: (1, 0, 0)),
            pl.BlockSpec((D, SHARED_DFF), lambda t: (0, 0)),
        ],
        out_specs=pl.BlockSpec((CTILE, D), lambda t: (t, 0)),
        out_shape=jax.ShapeDtypeStruct((S, D), jnp.float32),
        compiler_params=pltpu.CompilerParams(
            dimension_semantics=("arbitrary",)),
    )(rt, yg, xb, shW, shW, shared_down_W)
    return out.reshape(B, S, D)


_ORIG = kernel


# speedup vs baseline: 1.3091x; 1.0266x over previous
import jax
import jax.numpy as jnp
from jax.experimental import pallas as pl
from jax.experimental.pallas import tpu as pltpu
from jax.experimental.pallas import tpu_sc as plsc

D_MODEL = 768
NUM_EXPERTS = 8
ROUTED_DFF = 2304
SHARED_DFF = 768
SEQ = 2048

TILE = 256
NT = 24
GROWS = NT * TILE
GBLK = 512
NGB = GROWS // GBLK
CH = 768
NCH = ROUTED_DFF // CH
CTILE = 256
APW = (2 * SEQ) // 32


def _fiota(shape, dim):
    return jax.lax.broadcasted_iota(jnp.int32, shape, dim).astype(jnp.float32)


def _cumsum_rows(oh, tri):
    nblk = SEQ // GBLK
    outs = []
    carry = jnp.zeros((1, NUM_EXPERTS), jnp.float32)
    for b in range(nblk):
        blk = oh[b * GBLK:(b + 1) * GBLK, :]
        ex = jax.lax.dot_general(tri, blk.astype(jnp.bfloat16),
                                 (((1,), (0,)), ((), ())),
                                 preferred_element_type=jnp.float32)
        outs.append(ex + carry)
        carry = carry + jnp.sum(blk, axis=0, keepdims=True)
    return jnp.concatenate(outs, axis=0), carry


def _dispatch_kernel(x_ref, ln_scale_ref, ln_bias_ref, router_W_ref,
                     xn_out_ref, xb_out_ref, rt_ref, plan_ref):
    if True:
        x = x_ref[...]
        mu = jnp.mean(x, axis=1, keepdims=True)
        xc = x - mu
        var = jnp.mean(xc * xc, axis=1, keepdims=True)
        xn = xc * jax.lax.rsqrt(var + 1e-5)
        xn = xn * ln_scale_ref[...] + ln_bias_ref[...]
        xn_out_ref[...] = xn
        xb = xn.astype(jnp.bfloat16)
        xb_out_ref[...] = xb
        logits = jax.lax.dot_general(
            xb, router_W_ref[...].astype(jnp.bfloat16),
            (((1,), (1,)), ((), ())),
            preferred_element_type=jnp.float32)
        m = jnp.max(logits, axis=1, keepdims=True)
        ex = jnp.exp(logits - m)
        probs = ex / jnp.sum(ex, axis=1, keepdims=True)
        iota = _fiota(probs.shape, 1)
        p1 = jnp.max(probs, axis=1, keepdims=True)
        i1 = jnp.min(jnp.where(probs == p1, iota, NUM_EXPERTS), axis=1,
                     keepdims=True)
        masked = jnp.where(iota == i1, -1.0, probs)
        p2 = jnp.max(masked, axis=1, keepdims=True)
        i2 = jnp.min(jnp.where(masked == p2, iota, NUM_EXPERTS), axis=1,
                     keepdims=True)
        b = jnp.exp(p2 - p1)
        w1 = 1.0 / (1.0 + b)
        w2 = b / (1.0 + b)
        oh1 = (iota == i1).astype(jnp.float32)
        oh2 = (iota == i2).astype(jnp.float32)
        tri = (_fiota((GBLK, GBLK), 0) > _fiota((GBLK, GBLK), 1)
               ).astype(jnp.bfloat16)
        ex1, tot1 = _cumsum_rows(oh1, tri)
        ex2, tot2 = _cumsum_rows(oh2, tri)
        cnt = tot1 + tot2
        rank1 = jnp.sum(ex1 * oh1, axis=1, keepdims=True)
        rank2 = (jnp.sum(tot1 * oh2, axis=1, keepdims=True)
                 + jnp.sum(ex2 * oh2, axis=1, keepdims=True))
        ntiles = jnp.floor((cnt + (TILE - 1)) * (1.0 / TILE))
        sizes = ntiles * TILE
        tri8 = (_fiota((8, 8), 0) < _fiota((8, 8), 1)).astype(jnp.float32)
        offs = jax.lax.dot_general(sizes, tri8, (((1,), (0,)), ((), ())),
                                   precision=jax.lax.Precision.HIGHEST,
                                   preferred_element_type=jnp.float32)
        bounds = jax.lax.dot_general(
            ntiles,
            (_fiota((8, 8), 0) <= _fiota((8, 8), 1)).astype(jnp.float32),
            (((1,), (0,)), ((), ())),
            precision=jax.lax.Precision.HIGHEST,
            preferred_element_type=jnp.float32)
        pos1 = jnp.sum(offs * oh1, axis=1, keepdims=True) + rank1
        pos2 = jnp.sum(offs * oh2, axis=1, keepdims=True) + rank2
        rt = jnp.concatenate([pos1, pos2, w1, w2, i1, i2, jnp.zeros_like(p1),
                              jnp.zeros_like(p1)], axis=1)
        rt_ref[...] = rt
        tstart = bounds - ntiles
        plan_ref[...] = jnp.concatenate(
            [tstart, bounds, jnp.zeros((6, 8), jnp.float32)], axis=0)


def _sc_scatter_kernel(xn_hbm, idx_hbm, xg_hbm, idx_v, rows_v, sem):
    wid = (jax.lax.axis_index("s") * plsc.get_sparse_core_info().num_cores
           + jax.lax.axis_index("c"))
    base = wid * APW
    tbase = jax.lax.rem(base, SEQ)
    pltpu.sync_copy(idx_hbm.at[pl.ds(base, APW)], idx_v)
    pltpu.sync_copy(xn_hbm.at[pl.ds(tbase, APW), :], rows_v)
    pltpu.async_copy(rows_v, xg_hbm.at[idx_v], sem).wait()


def _expert_kernel(ts_ref, tn_ref, xg_ref, gate_ref, up_ref, down_ref,
                   yg_ref):
    e = pl.program_id(0)
    c = pl.program_id(1)

    @pl.when(jnp.logical_and(e == 0, c == 0))
    def _init():
        yg_ref[...] = jnp.zeros_like(yg_ref)

    gw = gate_ref[0, 0].astype(jnp.bfloat16)
    uw = up_ref[0, 0].astype(jnp.bfloat16)
    dw = down_ref[0].astype(jnp.bfloat16)

    def body(jt, _):
        r0 = jt * TILE
        xt = xg_ref[pl.ds(r0, TILE), :].astype(jnp.bfloat16)
        g = jax.lax.dot_general(xt, gw, (((1,), (1,)), ((), ())),
                                preferred_element_type=jnp.float32)
        u = jax.lax.dot_general(xt, uw, (((1,), (1,)), ((), ())),
                                preferred_element_type=jnp.float32)
        h = (g * jax.nn.sigmoid(g)) * u
        y = jax.lax.dot_general(h.astype(jnp.bfloat16), dw,
                                (((1,), (1,)), ((), ())),
                                preferred_element_type=jnp.float32)
        yg_ref[pl.ds(r0, TILE), :] += y
        return 0

    jax.lax.fori_loop(ts_ref[e], tn_ref[e], body, 0)


def _shared_kernel(xb_ref, sh_gate_ref, sh_up_ref, sh_down_ref, ys_ref):
    xb = xb_ref[...]
    sg = jax.lax.dot_general(xb, sh_gate_ref[0].astype(jnp.bfloat16),
                             (((1,), (1,)), ((), ())),
                             preferred_element_type=jnp.float32)
    su = jax.lax.dot_general(xb, sh_up_ref[0].astype(jnp.bfloat16),
                             (((1,), (1,)), ((), ())),
                             preferred_element_type=jnp.float32)
    sh = (sg * jax.nn.sigmoid(sg)) * su
    ys_ref[...] = jax.lax.dot_general(sh.astype(jnp.bfloat16),
                                      sh_down_ref[...].astype(jnp.bfloat16),
                                      (((1,), (1,)), ((), ())),
                                      preferred_element_type=jnp.float32)


def _combine_kernel(rt_ref, yg_ref, ys_ref, out_ref):
    pos1 = rt_ref[:, 0:1]
    pos2 = rt_ref[:, 1:2]
    w1 = rt_ref[:, 2:3]
    w2 = rt_ref[:, 3:4]
    lanes = _fiota((CTILE, GROWS), 1)
    comb = (jnp.where(lanes == pos1, w1, 0.0)
            + jnp.where(lanes == pos2, w2, 0.0)).astype(jnp.bfloat16)
    ygv = yg_ref[...]
    ygv = jnp.where(jnp.abs(ygv) <= 3.0e38, ygv, 0.0)
    routed = jax.lax.dot_general(comb, ygv.astype(jnp.bfloat16),
                                 (((1,), (0,)), ((), ())),
                                 preferred_element_type=jnp.float32)
    out_ref[...] = routed + ys_ref[...]


@jax.jit
def kernel(x, ln_scale, ln_bias, router_W, shared_gate_up_W, shared_down_W,
           expert_gate_up_W, expert_down_W):
    B, S, D = x.shape
    x2 = x.reshape(S, D)
    ln_scale2 = ln_scale.reshape(1, D)
    ln_bias2 = ln_bias.reshape(1, D)

    xn, xb, rt, plan = pl.pallas_call(
        _dispatch_kernel,
        grid=(1,),
        in_specs=[
            pl.BlockSpec((S, D), lambda p: (0, 0)),
            pl.BlockSpec((1, D), lambda p: (0, 0)),
            pl.BlockSpec((1, D), lambda p: (0, 0)),
            pl.BlockSpec((NUM_EXPERTS, D), lambda p: (0, 0)),
        ],
        out_specs=[
            pl.BlockSpec((S, D), lambda p: (0, 0)),
            pl.BlockSpec((S, D), lambda p: (0, 0)),
            pl.BlockSpec((S, 8), lambda p: (0, 0)),
            pl.BlockSpec((8, 8), lambda p: (0, 0)),
        ],
        out_shape=[
            jax.ShapeDtypeStruct((S, D), jnp.float32),
            jax.ShapeDtypeStruct((S, D), jnp.bfloat16),
            jax.ShapeDtypeStruct((S, 8), jnp.float32),
            jax.ShapeDtypeStruct((8, 8), jnp.float32),
        ],
        compiler_params=pltpu.CompilerParams(
            dimension_semantics=("arbitrary",)),
    )(x2, ln_scale2, ln_bias2, router_W)

    idx = jnp.concatenate([rt[:, 0], rt[:, 1]], axis=0).astype(jnp.int32)
    mesh = plsc.VectorSubcoreMesh(core_axis_name="c", subcore_axis_name="s")
    xg = pl.kernel(
        _sc_scatter_kernel,
        mesh=mesh,
        out_type=jax.ShapeDtypeStruct((GROWS, D), jnp.float32),
        scratch_types=[
            pltpu.VMEM((APW,), jnp.int32),
            pltpu.VMEM((APW, D), jnp.float32),
            pltpu.SemaphoreType.DMA,
        ],
    )(xn, idx)

    tstart = plan[0].astype(jnp.int32)
    tend = plan[1].astype(jnp.int32)

    gu4 = expert_gate_up_W.reshape(NUM_EXPERTS, 2 * NCH, CH, D)
    grid_spec = pltpu.PrefetchScalarGridSpec(
        num_scalar_prefetch=2,
        grid=(NUM_EXPERTS, NCH),
        in_specs=[
            pl.BlockSpec((GROWS, D), lambda e, c, ts, tn: (0, 0)),
            pl.BlockSpec((1, 1, CH, D), lambda e, c, ts, tn: (e, c, 0, 0)),
            pl.BlockSpec((1, 1, CH, D),
                         lambda e, c, ts, tn: (e, NCH + c, 0, 0)),
            pl.BlockSpec((1, D, CH), lambda e, c, ts, tn: (e, 0, c)),
        ],
        out_specs=pl.BlockSpec((GROWS, D), lambda e, c, ts, tn: (0, 0)),
    )
    yg = pl.pallas_call(
        _expert_kernel,
        grid_spec=grid_spec,
        out_shape=jax.ShapeDtypeStruct((GROWS, D), jnp.float32),
        compiler_params=pltpu.CompilerParams(
            dimension_semantics=("arbitrary", "arbitrary")),
    )(tstart, tend, xg, gu4, gu4, expert_down_W)

    shW = shared_gate_up_W.reshape(2, SHARED_DFF, D)
    ys = pl.pallas_call(
        _shared_kernel,
        grid=(S // CTILE,),
        in_specs=[
            pl.BlockSpec((CTILE, D), lambda t: (t, 0)),
            pl.BlockSpec((1, SHARED_DFF, D), lambda t: (0, 0, 0)),
            pl.BlockSpec((1, SHARED_DFF, D), lambda t: (1, 0, 0)),
            pl.BlockSpec((D, SHARED_DFF), lambda t: (0, 0)),
        ],
        out_specs=pl.BlockSpec((CTILE, D), lambda t: (t, 0)),
        out_shape=jax.ShapeDtypeStruct((S, D), jnp.float32),
        compiler_params=pltpu.CompilerParams(
            dimension_semantics=("arbitrary",)),
    )(xb, shW, shW, shared_down_W)

    out = pl.pallas_call(
        _combine_kernel,
        grid=(S // CTILE,),
        in_specs=[
            pl.BlockSpec((CTILE, 8), lambda t: (t, 0)),
            pl.BlockSpec((GROWS, D), lambda t: (0, 0)),
            pl.BlockSpec((CTILE, D), lambda t: (t, 0)),
        ],
        out_specs=pl.BlockSpec((CTILE, D), lambda t: (t, 0)),
        out_shape=jax.ShapeDtypeStruct((S, D), jnp.float32),
        compiler_params=pltpu.CompilerParams(
            dimension_semantics=("arbitrary",)),
    )(rt, yg, ys)
    return out.reshape(B, S, D)


_ORIG = kernel


# speedup vs baseline: 1.3114x; 1.0018x over previous
import jax
import jax.numpy as jnp
from jax.experimental import pallas as pl
from jax.experimental.pallas import tpu as pltpu
from jax.experimental.pallas import tpu_sc as plsc

D_MODEL = 768
NUM_EXPERTS = 8
ROUTED_DFF = 2304
SHARED_DFF = 768
SEQ = 2048

TILE = 256
NT = 24
GROWS = NT * TILE
GBLK = 512
CH = 768
NCH = ROUTED_DFF // CH
CTILE = 256
APW = (2 * SEQ) // 32


def _fiota(shape, dim):
    return jax.lax.broadcasted_iota(jnp.int32, shape, dim).astype(jnp.float32)


def _cumsum_rows(oh, tri):
    nblk = SEQ // GBLK
    outs = []
    carry = jnp.zeros((1, NUM_EXPERTS), jnp.float32)
    for b in range(nblk):
        blk = oh[b * GBLK:(b + 1) * GBLK, :]
        ex = jax.lax.dot_general(tri, blk.astype(jnp.bfloat16),
                                 (((1,), (0,)), ((), ())),
                                 preferred_element_type=jnp.float32)
        outs.append(ex + carry)
        carry = carry + jnp.sum(blk, axis=0, keepdims=True)
    return jnp.concatenate(outs, axis=0), carry


def _dispatch_kernel(x_ref, ln_scale_ref, ln_bias_ref, router_W_ref,
                     xn_out_ref, xb_out_ref, rt_ref, plan_ref):
    if True:
        x = x_ref[...]
        mu = jnp.mean(x, axis=1, keepdims=True)
        xc = x - mu
        var = jnp.mean(xc * xc, axis=1, keepdims=True)
        xn = xc * jax.lax.rsqrt(var + 1e-5)
        xn = xn * ln_scale_ref[...] + ln_bias_ref[...]
        xn_out_ref[...] = xn
        xb = xn.astype(jnp.bfloat16)
        xb_out_ref[...] = xb
        logits = jax.lax.dot_general(
            xb, router_W_ref[...].astype(jnp.bfloat16),
            (((1,), (1,)), ((), ())),
            preferred_element_type=jnp.float32)
        m = jnp.max(logits, axis=1, keepdims=True)
        ex = jnp.exp(logits - m)
        probs = ex / jnp.sum(ex, axis=1, keepdims=True)
        iota = _fiota(probs.shape, 1)
        p1 = jnp.max(probs, axis=1, keepdims=True)
        i1 = jnp.min(jnp.where(probs == p1, iota, NUM_EXPERTS), axis=1,
                     keepdims=True)
        masked = jnp.where(iota == i1, -1.0, probs)
        p2 = jnp.max(masked, axis=1, keepdims=True)
        i2 = jnp.min(jnp.where(masked == p2, iota, NUM_EXPERTS), axis=1,
                     keepdims=True)
        b = jnp.exp(p2 - p1)
        w1 = 1.0 / (1.0 + b)
        w2 = b / (1.0 + b)
        oh1 = (iota == i1).astype(jnp.float32)
        oh2 = (iota == i2).astype(jnp.float32)
        tri = (_fiota((GBLK, GBLK), 0) > _fiota((GBLK, GBLK), 1)
               ).astype(jnp.bfloat16)
        ex1, tot1 = _cumsum_rows(oh1, tri)
        ex2, tot2 = _cumsum_rows(oh2, tri)
        cnt = tot1 + tot2
        rank1 = jnp.sum(ex1 * oh1, axis=1, keepdims=True)
        rank2 = (jnp.sum(tot1 * oh2, axis=1, keepdims=True)
                 + jnp.sum(ex2 * oh2, axis=1, keepdims=True))
        ntiles = jnp.floor((cnt + (TILE - 1)) * (1.0 / TILE))
        sizes = ntiles * TILE
        tri8 = (_fiota((8, 8), 0) < _fiota((8, 8), 1)).astype(jnp.float32)
        offs = jax.lax.dot_general(sizes, tri8, (((1,), (0,)), ((), ())),
                                   precision=jax.lax.Precision.HIGHEST,
                                   preferred_element_type=jnp.float32)
        bounds = jax.lax.dot_general(
            ntiles,
            (_fiota((8, 8), 0) <= _fiota((8, 8), 1)).astype(jnp.float32),
            (((1,), (0,)), ((), ())),
            precision=jax.lax.Precision.HIGHEST,
            preferred_element_type=jnp.float32)
        pos1 = jnp.sum(offs * oh1, axis=1, keepdims=True) + rank1
        pos2 = jnp.sum(offs * oh2, axis=1, keepdims=True) + rank2
        rt = jnp.concatenate([pos1, pos2, w1, w2, i1, i2, jnp.zeros_like(p1),
                              jnp.zeros_like(p1)], axis=1)
        rt_ref[...] = rt
        tstart = bounds - ntiles
        plan_ref[...] = jnp.concatenate(
            [tstart, bounds, jnp.zeros((6, 8), jnp.float32)], axis=0)


def _sc_scatter_kernel(xn_hbm, idx_hbm, xg_hbm, idx_v, rows_v, sem):
    wid = (jax.lax.axis_index("s") * plsc.get_sparse_core_info().num_cores
           + jax.lax.axis_index("c"))
    base = wid * APW
    tbase = jax.lax.rem(base, SEQ)
    pltpu.sync_copy(idx_hbm.at[pl.ds(base, APW)], idx_v)
    pltpu.sync_copy(xn_hbm.at[pl.ds(tbase, APW), :], rows_v)
    pltpu.async_copy(rows_v, xg_hbm.at[idx_v], sem).wait()


def _expert_kernel(ts_ref, tn_ref, xg_ref, gate_ref, up_ref, down_ref,
                   yg_ref):
    e = pl.program_id(0)
    c = pl.program_id(1)

    @pl.when(jnp.logical_and(e == 0, c == 0))
    def _init():
        yg_ref[...] = jnp.zeros_like(yg_ref)

    gw = gate_ref[0, 0].astype(jnp.bfloat16)
    uw = up_ref[0, 0].astype(jnp.bfloat16)
    dw = down_ref[0].astype(jnp.bfloat16)

    def body(jt, _):
        r0 = jt * TILE
        xt = xg_ref[pl.ds(r0, TILE), :].astype(jnp.bfloat16)
        g = jax.lax.dot_general(xt, gw, (((1,), (1,)), ((), ())),
                                preferred_element_type=jnp.float32)
        u = jax.lax.dot_general(xt, uw, (((1,), (1,)), ((), ())),
                                preferred_element_type=jnp.float32)
        h = (g * jax.nn.sigmoid(g)) * u
        y = jax.lax.dot_general(h.astype(jnp.bfloat16), dw,
                                (((1,), (1,)), ((), ())),
                                preferred_element_type=jnp.float32)
        yg_ref[pl.ds(r0, TILE), :] += y
        return 0

    jax.lax.fori_loop(ts_ref[e], tn_ref[e], body, 0)


def _shared_kernel(xb_ref, sh_gate_ref, sh_up_ref, sh_down_ref, ys_ref):
    xb = xb_ref[...]
    sg = jax.lax.dot_general(xb, sh_gate_ref[0].astype(jnp.bfloat16),
                             (((1,), (1,)), ((), ())),
                             preferred_element_type=jnp.float32)
    su = jax.lax.dot_general(xb, sh_up_ref[0].astype(jnp.bfloat16),
                             (((1,), (1,)), ((), ())),
                             preferred_element_type=jnp.float32)
    sh = (sg * jax.nn.sigmoid(sg)) * su
    ys_ref[...] = jax.lax.dot_general(sh.astype(jnp.bfloat16),
                                      sh_down_ref[...].astype(jnp.bfloat16),
                                      (((1,), (1,)), ((), ())),
                                      preferred_element_type=jnp.float32)


def _combine_kernel(rt_ref, yg_ref, ys_ref, out_ref):
    pos1 = rt_ref[:, 0:1]
    pos2 = rt_ref[:, 1:2]
    w1 = rt_ref[:, 2:3]
    w2 = rt_ref[:, 3:4]
    lanes = _fiota((CTILE, GROWS), 1)
    comb = (jnp.where(lanes == pos1, w1, 0.0)
            + jnp.where(lanes == pos2, w2, 0.0)).astype(jnp.bfloat16)
    ygv = yg_ref[...]
    ygv = jnp.where(jnp.abs(ygv) <= 3.0e38, ygv, 0.0)
    routed = jax.lax.dot_general(comb, ygv.astype(jnp.bfloat16),
                                 (((1,), (0,)), ((), ())),
                                 preferred_element_type=jnp.float32)
    out_ref[...] = routed + ys_ref[...]


@jax.jit
def kernel(x, ln_scale, ln_bias, router_W, shared_gate_up_W, shared_down_W,
           expert_gate_up_W, expert_down_W):
    B, S, D = x.shape
    x2 = x.reshape(S, D)
    ln_scale2 = ln_scale.reshape(1, D)
    ln_bias2 = ln_bias.reshape(1, D)

    xn, xb, rt, plan = pl.pallas_call(
        _dispatch_kernel,
        grid=(1,),
        in_specs=[
            pl.BlockSpec((S, D), lambda p: (0, 0)),
            pl.BlockSpec((1, D), lambda p: (0, 0)),
            pl.BlockSpec((1, D), lambda p: (0, 0)),
            pl.BlockSpec((NUM_EXPERTS, D), lambda p: (0, 0)),
        ],
        out_specs=[
            pl.BlockSpec((S, D), lambda p: (0, 0)),
            pl.BlockSpec((S, D), lambda p: (0, 0)),
            pl.BlockSpec((S, 8), lambda p: (0, 0)),
            pl.BlockSpec((8, 8), lambda p: (0, 0)),
        ],
        out_shape=[
            jax.ShapeDtypeStruct((S, D), jnp.float32),
            jax.ShapeDtypeStruct((S, D), jnp.bfloat16),
            jax.ShapeDtypeStruct((S, 8), jnp.float32),
            jax.ShapeDtypeStruct((8, 8), jnp.float32),
        ],
        compiler_params=pltpu.CompilerParams(
            dimension_semantics=("arbitrary",)),
    )(x2, ln_scale2, ln_bias2, router_W)

    idx = jnp.concatenate([rt[:, 0], rt[:, 1]], axis=0).astype(jnp.int32)
    mesh = plsc.VectorSubcoreMesh(core_axis_name="c", subcore_axis_name="s")
    xg = pl.kernel(
        _sc_scatter_kernel,
        mesh=mesh,
        out_type=jax.ShapeDtypeStruct((GROWS, D), jnp.float32),
        scratch_types=[
            pltpu.VMEM((APW,), jnp.int32),
            pltpu.VMEM((APW, D), jnp.float32),
            pltpu.SemaphoreType.DMA,
        ],
    )(xn, idx)

    tstart = plan[0].astype(jnp.int32)
    tend = plan[1].astype(jnp.int32)

    gu4 = expert_gate_up_W.reshape(NUM_EXPERTS, 2 * NCH, CH, D)
    grid_spec = pltpu.PrefetchScalarGridSpec(
        num_scalar_prefetch=2,
        grid=(NUM_EXPERTS, NCH),
        in_specs=[
            pl.BlockSpec((GROWS, D), lambda e, c, ts, tn: (0, 0)),
            pl.BlockSpec((1, 1, CH, D), lambda e, c, ts, tn: (e, c, 0, 0)),
            pl.BlockSpec((1, 1, CH, D),
                         lambda e, c, ts, tn: (e, NCH + c, 0, 0)),
            pl.BlockSpec((1, D, CH), lambda e, c, ts, tn: (e, 0, c)),
        ],
        out_specs=pl.BlockSpec((GROWS, D), lambda e, c, ts, tn: (0, 0)),
    )
    yg = pl.pallas_call(
        _expert_kernel,
        grid_spec=grid_spec,
        out_shape=jax.ShapeDtypeStruct((GROWS, D), jnp.float32),
        compiler_params=pltpu.CompilerParams(
            dimension_semantics=("arbitrary", "arbitrary")),
    )(tstart, tend, xg, gu4, gu4, expert_down_W)

    shW = shared_gate_up_W.reshape(2, SHARED_DFF, D)
    ys = pl.pallas_call(
        _shared_kernel,
        grid=(S // CTILE,),
        in_specs=[
            pl.BlockSpec((CTILE, D), lambda t: (t, 0)),
            pl.BlockSpec((1, SHARED_DFF, D), lambda t: (0, 0, 0)),
            pl.BlockSpec((1, SHARED_DFF, D), lambda t: (1, 0, 0)),
            pl.BlockSpec((D, SHARED_DFF), lambda t: (0, 0)),
        ],
        out_specs=pl.BlockSpec((CTILE, D), lambda t: (t, 0)),
        out_shape=jax.ShapeDtypeStruct((S, D), jnp.float32),
        compiler_params=pltpu.CompilerParams(
            dimension_semantics=("arbitrary",)),
    )(xb, shW, shW, shared_down_W)

    out = pl.pallas_call(
        _combine_kernel,
        grid=(S // CTILE,),
        in_specs=[
            pl.BlockSpec((CTILE, 8), lambda t: (t, 0)),
            pl.BlockSpec((GROWS, D), lambda t: (0, 0)),
            pl.BlockSpec((CTILE, D), lambda t: (t, 0)),
        ],
        out_specs=pl.BlockSpec((CTILE, D), lambda t: (t, 0)),
        out_shape=jax.ShapeDtypeStruct((S, D), jnp.float32),
        compiler_params=pltpu.CompilerParams(
            dimension_semantics=("arbitrary",)),
    )(rt, yg, ys)
    return out.reshape(B, S, D)


_ORIG = kernel
